# baseline (speedup 1.0000x reference)
import functools

import jax
import jax.numpy as jnp
from jax import lax
from jax.experimental import pallas as pl
from jax.experimental.pallas import tpu as pltpu
from jax.experimental.pallas import tpu_sc as plsc

_T = 16
_S = 16
_ST = _S * _T
_G = 64
_N = 10000
_E = 160000
_F = 128

_NH_BLK = 2000
_CHUNK = 128
_NCHUNK = _E // _CHUNK
_NTILES = 32
_CPT = 40

_NP = 10240
_NODE_BLK = 2048
_EDGE_BLK = 3200


def _nh_body(x_ref, v_ref, o_ref):
    o_ref[...] = lax.dot_general(
        x_ref[...], v_ref[...], (((1,), (0,)), ((), ())),
        preferred_element_type=jnp.float32)


def _compute_nh(x, v):
    return pl.pallas_call(
        _nh_body,
        grid=(_N // _NH_BLK,),
        in_specs=[
            pl.BlockSpec((_NH_BLK, _F), lambda i: (i, 0)),
            pl.BlockSpec((_F, _T), lambda i: (0, 0)),
        ],
        out_specs=pl.BlockSpec((_NH_BLK, _T), lambda i: (i, 0)),
        out_shape=jax.ShapeDtypeStruct((_N, _T), jnp.float32),
    )(x, v)


def _sc_edge_gather(nh, ei, batch, n_edges, cpt, base_cnt, n_extra):
    mesh = plsc.VectorSubcoreMesh(core_axis_name="c", subcore_axis_name="s")

    @functools.partial(
        pl.kernel,
        mesh=mesh,
        compiler_params=pltpu.CompilerParams(
            needs_layout_passes=False, use_tc_tiling_on_sc=False),
        out_type=jax.ShapeDtypeStruct((_T + 1, n_edges), jnp.float32),
        scratch_types=[
            pltpu.VMEM((2, cpt * _CHUNK), jnp.int32),
            pltpu.VMEM((_CHUNK, _T), jnp.float32),
            pltpu.VMEM((_CHUNK, _T), jnp.float32),
            pltpu.VMEM((_CHUNK, _T), jnp.float32),
            pltpu.VMEM((_CHUNK, _T), jnp.float32),
            pltpu.VMEM((_T + 1, _CHUNK), jnp.float32),
            pltpu.VMEM((_T + 1, _CHUNK), jnp.float32),
            pltpu.VMEM((_N,), jnp.int32),
            pltpu.SemaphoreType.DMA,
            pltpu.SemaphoreType.DMA,
            pltpu.SemaphoreType.DMA,
            pltpu.SemaphoreType.DMA,
            pltpu.SemaphoreType.DMA,
            pltpu.SemaphoreType.DMA,
            pltpu.SemaphoreType.DMA,
        ],
    )
    def k(nh_hbm, ei_hbm, b_hbm, out_hbm,
          idx_all, ra0, rb0, ra1, rb1, et0, et1, batch_v,
          sga0, sgb0, so0, sga1, sgb1, so1, si):
        wid = lax.axis_index("s") * 2 + lax.axis_index("c")
        start = base_cnt * wid + jnp.maximum(wid - (_NTILES - n_extra), 0)
        cnt = jnp.where(wid >= _NTILES - n_extra, base_cnt + 1, base_cnt)
        base = start * _CHUNK
        pltpu.async_copy(ei_hbm.at[:, pl.ds(base, cpt * _CHUNK)], idx_all, si)
        pltpu.sync_copy(b_hbm, batch_v)
        pltpu.make_async_copy(
            ei_hbm.at[:, pl.ds(base, cpt * _CHUNK)], idx_all, si).wait()
        lanes = lax.iota(jnp.int32, 16)
        slots = ((ra0, rb0, et0, sga0, sgb0, so0),
                 (ra1, rb1, et1, sga1, sgb1, so1))

        def gath(kk, ra, rb, sga, sgb):
            off = kk * _CHUNK
            pltpu.async_copy(nh_hbm.at[idx_all.at[0, pl.ds(off, _CHUNK)]], ra, sga)
            pltpu.async_copy(nh_hbm.at[idx_all.at[1, pl.ds(off, _CHUNK)]], rb, sgb)

        for b in (0, 1):
            ra, rb, _, sga, sgb, _ = slots[b]
            gath(jnp.int32(b), ra, rb, sga, sgb)

        def step(i, carry):
            for b in (0, 1):
                ra, rb, et, sga, sgb, so = slots[b]
                kk = 2 * i + b

                @pl.when(kk < cnt)
                def _():
                    cb = (start + kk) * _CHUNK
                    off = kk * _CHUNK
                    pltpu.make_async_copy(
                        nh_hbm.at[idx_all.at[0, pl.ds(off, _CHUNK)]], ra, sga).wait()
                    pltpu.make_async_copy(
                        nh_hbm.at[idx_all.at[1, pl.ds(off, _CHUNK)]], rb, sgb).wait()

                    @pl.when(kk >= 2)
                    def _():
                        pltpu.make_async_copy(
                            et, out_hbm.at[:, pl.ds(cb, _CHUNK)], so).wait()

                    def compute(i16, c):
                        g16 = plsc.load_gather(
                            batch_v, [idx_all[0, pl.ds(off + i16 * 16, 16)]])
                        et[16, pl.ds(i16 * 16, 16)] = -(g16.astype(jnp.float32) + 0.25)
                        for j in range(16):
                            r = i16 * 16 + j
                            eh = jnp.maximum(ra[r, :], rb[r, :])
                            plsc.store_scatter(
                                et, [lanes, jnp.full((16,), r, jnp.int32)], eh)
                        return c
                    lax.fori_loop(0, _CHUNK // 16, compute, 0)

                    pltpu.async_copy(et, out_hbm.at[:, pl.ds(cb, _CHUNK)], so)

                    @pl.when(kk + 2 < cnt)
                    def _():
                        gath(kk + 2, ra, rb, sga, sgb)
            return carry
        lax.fori_loop(0, (cpt + 1) // 2, step, 0)

        for b in (0, 1):
            _, _, et, _, _, so = slots[b]
            pltpu.make_async_copy(
                et, out_hbm.at[:, pl.ds(base, _CHUNK)], so).wait()

    return k(nh, ei, batch)


def _ecc_body(blk, y_ref, o_ref):
    yb = y_ref[0:16, :]
    gw = y_ref[16:17, :]
    yt = jnp.concatenate([yb] * _S, axis=0)
    srow = lax.broadcasted_iota(jnp.int32, (_ST, blk), 0) >> 4
    lin = srow.astype(jnp.float32) * jnp.float32(2.0 / 15.0) - 1.0
    sig = 0.5 + 0.5 * jnp.tanh(100.0 * (lin - yt))

    gbc = jnp.broadcast_to(gw, (_G, blk))
    iot = lax.broadcasted_iota(jnp.int32, (_G, blk), 0)
    oh = jnp.where(jnp.abs(gbc).astype(jnp.int32) == iot,
                   jnp.sign(gbc), 0.0)
    d = lax.dot_general(
        sig.astype(jnp.bfloat16), oh.astype(jnp.bfloat16),
        (((1,), (1,)), ((), ())), preferred_element_type=jnp.float32)

    @pl.when(pl.program_id(0) == 0)
    def _():
        o_ref[...] = d

    @pl.when(pl.program_id(0) != 0)
    def _():
        o_ref[...] += d


def _compute_ecc(y17, blk):
    n = y17.shape[1]
    return pl.pallas_call(
        functools.partial(_ecc_body, blk),
        grid=(n // blk,),
        in_specs=[pl.BlockSpec((_T + 1, blk), lambda i: (0, i))],
        out_specs=pl.BlockSpec((_ST, _G), lambda i: (0, 0)),
        out_shape=jax.ShapeDtypeStruct((_ST, _G), jnp.float32),
    )(y17)


def kernel(x, edge_index, batch, v):
    nh = _compute_nh(x, v)
    quarters = []
    off = 0
    for n_e, base_cnt, n_extra in ((41600, 10, 5), (41600, 10, 5),
                                   (38400, 9, 12), (38400, 9, 12)):
        quarters.append(_sc_edge_gather(
            nh, edge_index[:, off:off + n_e], batch,
            n_e, base_cnt + 1, base_cnt, n_extra))
        off += n_e

    n17 = jnp.concatenate(
        [nh.T, (batch.astype(jnp.float32) + 0.25)[None, :]], axis=0)
    n17p = jnp.pad(n17, ((0, 0), (0, _NP - _N)))

    out2 = _compute_ecc(n17p, _NODE_BLK)
    for q in quarters:
        out2 = out2 + _compute_ecc(q, _EDGE_BLK)
    return out2.T.reshape(_G, _S, _T)


# speedup vs baseline: 1.0486x; 1.0486x over previous
import functools

import jax
import jax.numpy as jnp
from jax import lax
from jax.experimental import pallas as pl
from jax.experimental.pallas import tpu as pltpu
from jax.experimental.pallas import tpu_sc as plsc

_T = 16
_S = 16
_ST = _S * _T
_G = 64
_N = 10000
_E = 160000
_F = 128

_NH_BLK = 2000
_CHUNK = 128
_NCHUNK = _E // _CHUNK
_NTILES = 32
_CPT = 40

_NP = 10240
_NODE_BLK = 2048
_EDGE_BLK = 3200


def _nh_body(x_ref, v_ref, o_ref):
    o_ref[...] = lax.dot_general(
        x_ref[...], v_ref[...], (((1,), (0,)), ((), ())),
        preferred_element_type=jnp.float32)


def _compute_nh(x, v):
    return pl.pallas_call(
        _nh_body,
        grid=(_N // _NH_BLK,),
        in_specs=[
            pl.BlockSpec((_NH_BLK, _F), lambda i: (i, 0)),
            pl.BlockSpec((_F, _T), lambda i: (0, 0)),
        ],
        out_specs=pl.BlockSpec((_NH_BLK, _T), lambda i: (i, 0)),
        out_shape=jax.ShapeDtypeStruct((_N, _T), jnp.float32),
    )(x, v)


def _sc_edge_gather(nh, ei, batch, n_edges, cpt, base_cnt, n_extra):
    mesh = plsc.VectorSubcoreMesh(core_axis_name="c", subcore_axis_name="s")

    @functools.partial(
        pl.kernel,
        mesh=mesh,
        compiler_params=pltpu.CompilerParams(
            needs_layout_passes=False, use_tc_tiling_on_sc=False),
        out_type=jax.ShapeDtypeStruct((_T + 1, n_edges), jnp.float32),
        scratch_types=[
            pltpu.VMEM((2, cpt * _CHUNK), jnp.int32),
            pltpu.VMEM((_CHUNK, _T), jnp.float32),
            pltpu.VMEM((_CHUNK, _T), jnp.float32),
            pltpu.VMEM((_CHUNK, _T), jnp.float32),
            pltpu.VMEM((_CHUNK, _T), jnp.float32),
            pltpu.VMEM((_T + 1, _CHUNK), jnp.float32),
            pltpu.VMEM((_T + 1, _CHUNK), jnp.float32),
            pltpu.VMEM((_N,), jnp.int32),
            pltpu.SemaphoreType.DMA,
            pltpu.SemaphoreType.DMA,
            pltpu.SemaphoreType.DMA,
            pltpu.SemaphoreType.DMA,
            pltpu.SemaphoreType.DMA,
            pltpu.SemaphoreType.DMA,
            pltpu.SemaphoreType.DMA,
        ],
    )
    def k(nh_hbm, ei_hbm, b_hbm, out_hbm,
          idx_all, ra0, rb0, ra1, rb1, et0, et1, batch_v,
          sga0, sgb0, so0, sga1, sgb1, so1, si):
        wid = lax.axis_index("s") * 2 + lax.axis_index("c")
        start = base_cnt * wid + jnp.maximum(wid - (_NTILES - n_extra), 0)
        cnt = jnp.where(wid >= _NTILES - n_extra, base_cnt + 1, base_cnt)
        base = start * _CHUNK
        pltpu.async_copy(ei_hbm.at[:, pl.ds(base, cpt * _CHUNK)], idx_all, si)
        pltpu.sync_copy(b_hbm, batch_v)
        pltpu.make_async_copy(
            ei_hbm.at[:, pl.ds(base, cpt * _CHUNK)], idx_all, si).wait()
        lanes = lax.iota(jnp.int32, 16)
        slots = ((ra0, rb0, et0, sga0, sgb0, so0),
                 (ra1, rb1, et1, sga1, sgb1, so1))

        def gath(kk, ra, rb, sga, sgb):
            off = kk * _CHUNK
            pltpu.async_copy(nh_hbm.at[idx_all.at[0, pl.ds(off, _CHUNK)]], ra, sga)
            pltpu.async_copy(nh_hbm.at[idx_all.at[1, pl.ds(off, _CHUNK)]], rb, sgb)

        for b in (0, 1):
            ra, rb, _, sga, sgb, _ = slots[b]
            gath(jnp.int32(b), ra, rb, sga, sgb)

        def step(i, carry):
            for b in (0, 1):
                ra, rb, et, sga, sgb, so = slots[b]
                kk = 2 * i + b

                @pl.when(kk < cnt)
                def _():
                    cb = (start + kk) * _CHUNK
                    off = kk * _CHUNK
                    pltpu.make_async_copy(
                        nh_hbm.at[idx_all.at[0, pl.ds(off, _CHUNK)]], ra, sga).wait()
                    pltpu.make_async_copy(
                        nh_hbm.at[idx_all.at[1, pl.ds(off, _CHUNK)]], rb, sgb).wait()

                    @pl.when(kk >= 2)
                    def _():
                        pltpu.make_async_copy(
                            et, out_hbm.at[:, pl.ds(cb, _CHUNK)], so).wait()

                    def compute(i16, c):
                        g16 = plsc.load_gather(
                            batch_v, [idx_all[0, pl.ds(off + i16 * 16, 16)]])
                        et[16, pl.ds(i16 * 16, 16)] = -(g16.astype(jnp.float32) + 0.25)
                        for j in range(16):
                            r = i16 * 16 + j
                            eh = jnp.maximum(ra[r, :], rb[r, :])
                            plsc.store_scatter(
                                et, [lanes, jnp.full((16,), r, jnp.int32)], eh)
                        return c
                    lax.fori_loop(0, _CHUNK // 16, compute, 0)

                    pltpu.async_copy(et, out_hbm.at[:, pl.ds(cb, _CHUNK)], so)

                    @pl.when(kk + 2 < cnt)
                    def _():
                        gath(kk + 2, ra, rb, sga, sgb)
            return carry
        lax.fori_loop(0, (cpt + 1) // 2, step, 0)

        for b in (0, 1):
            _, _, et, _, _, so = slots[b]
            pltpu.make_async_copy(
                et, out_hbm.at[:, pl.ds(base, _CHUNK)], so).wait()

    return k(nh, ei, batch)


def _ecc_body(blk, y_ref, o_ref):
    yb = y_ref[0:16, :]
    gw = y_ref[16:17, :]
    yt = jnp.concatenate([yb] * _S, axis=0)
    srow = lax.broadcasted_iota(jnp.int32, (_ST, blk), 0) >> 4
    lin = srow.astype(jnp.float32) * jnp.float32(2.0 / 15.0) - 1.0
    sig = 0.5 + 0.5 * jnp.tanh(100.0 * (lin - yt))

    gbc = jnp.broadcast_to(gw, (_G, blk))
    iot = lax.broadcasted_iota(jnp.int32, (_G, blk), 0)
    oh = jnp.where(jnp.abs(gbc).astype(jnp.int32) == iot,
                   jnp.sign(gbc), 0.0)
    d = lax.dot_general(
        sig.astype(jnp.bfloat16), oh.astype(jnp.bfloat16),
        (((1,), (1,)), ((), ())), preferred_element_type=jnp.float32)

    @pl.when(pl.program_id(0) == 0)
    def _():
        o_ref[...] = d

    @pl.when(pl.program_id(0) != 0)
    def _():
        o_ref[...] += d


def _compute_ecc(y17, blk):
    n = y17.shape[1]
    return pl.pallas_call(
        functools.partial(_ecc_body, blk),
        grid=(n // blk,),
        in_specs=[pl.BlockSpec((_T + 1, blk), lambda i: (0, i))],
        out_specs=pl.BlockSpec((_ST, _G), lambda i: (0, 0)),
        out_shape=jax.ShapeDtypeStruct((_ST, _G), jnp.float32),
    )(y17)


def kernel(x, edge_index, batch, v):
    nh = _compute_nh(x, v)
    eh = _E // 2
    ea = _sc_edge_gather(nh, edge_index[:, :eh], batch, eh, 20, 19, 17)
    eb = _sc_edge_gather(nh, edge_index[:, eh:], batch, eh, 20, 19, 17)

    n17 = jnp.concatenate(
        [nh.T, (batch.astype(jnp.float32) + 0.25)[None, :]], axis=0)
    n17p = jnp.pad(n17, ((0, 0), (0, _NP - _N)))

    out2 = (_compute_ecc(n17p, _NODE_BLK)
            + _compute_ecc(ea, _EDGE_BLK)
            + _compute_ecc(eb, _EDGE_BLK))
    return out2.T.reshape(_G, _S, _T)
